# speedup vs baseline: 2.4103x; 2.4103x over previous
import functools
import math

import jax
import jax.numpy as jnp
from jax import lax
from jax.experimental import pallas as pl
from jax.experimental.pallas import tpu as pltpu
from jax.experimental.pallas import tpu_sc as plsc

VOCAB = 100000
D = 128
BATCH = 4096
SEQ = 50
TOTAL = BATCH * SEQ
NW = 32
ROWS_PER_W = TOTAL // NW
CHUNK = 128
N_CHUNKS = ROWS_PER_W // CHUNK
LANES = 16
SCALE = math.sqrt(D)

_mesh = plsc.VectorSubcoreMesh(core_axis_name="c", subcore_axis_name="s")


@functools.partial(
    pl.kernel,
    out_type=jax.ShapeDtypeStruct((TOTAL, D), jnp.float32),
    mesh=_mesh,
    scratch_types=[
        pltpu.VMEM((ROWS_PER_W,), jnp.int32),
        pltpu.VMEM((CHUNK, D), jnp.float32),
        pltpu.SemaphoreType.DMA,
    ],
)
def _embed_sc(idx_hbm, table_hbm, out_hbm, idx_v, rows_v, sem):
    wid = lax.axis_index("s") * 2 + lax.axis_index("c")
    base = wid * ROWS_PER_W
    pltpu.sync_copy(idx_hbm.at[pl.ds(base, ROWS_PER_W)], idx_v)

    def chunk_body(c, _):
        idx_slice = idx_v.at[pl.ds(c * CHUNK, CHUNK)]
        pltpu.async_copy(table_hbm.at[idx_slice], rows_v, sem).wait()

        def scale_row(r, _):
            for j in range(D // LANES):
                sl = pl.ds(j * LANES, LANES)
                rows_v[r, sl] = rows_v[r, sl] * SCALE
            return 0

        lax.fori_loop(0, CHUNK, scale_row, 0)
        pltpu.sync_copy(rows_v, out_hbm.at[pl.ds(base + c * CHUNK, CHUNK)])
        return 0

    lax.fori_loop(0, N_CHUNKS, chunk_body, 0)


def kernel(input_ids, embed_weight):
    idx = input_ids.reshape(TOTAL).astype(jnp.int32)
    out = _embed_sc(idx, embed_weight)
    return out.reshape(BATCH, SEQ, D)


# speedup vs baseline: 2.9543x; 1.2257x over previous
import functools
import math

import jax
import jax.numpy as jnp
from jax import lax
from jax.experimental import pallas as pl
from jax.experimental.pallas import tpu as pltpu
from jax.experimental.pallas import tpu_sc as plsc

VOCAB = 100000
D = 128
BATCH = 4096
SEQ = 50
TOTAL = BATCH * SEQ
NW = 32
ROWS_PER_W = TOTAL // NW
CHUNK = 64
N_CHUNKS = ROWS_PER_W // CHUNK
NBUF = 4
N_GROUPS = N_CHUNKS // NBUF
LANES = 16
SCALE = math.sqrt(D)

_mesh = plsc.VectorSubcoreMesh(core_axis_name="c", subcore_axis_name="s")


@functools.partial(
    pl.kernel,
    out_type=jax.ShapeDtypeStruct((TOTAL, D), jnp.float32),
    mesh=_mesh,
    scratch_types=[
        pltpu.VMEM((ROWS_PER_W,), jnp.int32),
        pltpu.VMEM((NBUF, CHUNK, D), jnp.float32),
        pltpu.VMEM((NBUF, CHUNK, D), jnp.float32),
    ]
    + [pltpu.SemaphoreType.DMA] * (2 * NBUF),
)
def _embed_sc(idx_hbm, table_hbm, out_hbm, idx_v, gbuf, sbuf, *sems):
    gsem = sems[:NBUF]
    ssem = sems[NBUF:]
    wid = lax.axis_index("s") * 2 + lax.axis_index("c")
    base = wid * ROWS_PER_W
    pltpu.sync_copy(idx_hbm.at[pl.ds(base, ROWS_PER_W)], idx_v)

    def gather_desc(c, b):
        return pltpu.make_async_copy(
            table_hbm.at[idx_v.at[pl.ds(c * CHUNK, CHUNK)]], gbuf.at[b], gsem[b])

    def store_desc(c, b):
        return pltpu.make_async_copy(
            sbuf.at[b], out_hbm.at[pl.ds(base + c * CHUNK, CHUNK)], ssem[b])

    for b in range(NBUF):
        gather_desc(b, b).start()

    def group(g, _):
        for b in range(NBUF):
            c = g * NBUF + b
            gather_desc(c, b).wait()

            @pl.when(g > 0)
            def _():
                store_desc(c - NBUF, b).wait()

            def scale_row(r, _):
                for j in range(D // LANES):
                    sl = pl.ds(j * LANES, LANES)
                    sbuf[b, r, sl] = gbuf[b, r, sl] * SCALE
                return 0

            lax.fori_loop(0, CHUNK, scale_row, 0)

            @pl.when(c + NBUF < N_CHUNKS)
            def _():
                gather_desc(c + NBUF, b).start()

            store_desc(c, b).start()
        return 0

    lax.fori_loop(0, N_GROUPS, group, 0)

    for b in range(NBUF):
        store_desc(N_CHUNKS - NBUF + b, b).wait()


def kernel(input_ids, embed_weight):
    idx = input_ids.reshape(TOTAL).astype(jnp.int32)
    out = _embed_sc(idx, embed_weight)
    return out.reshape(BATCH, SEQ, D)


# speedup vs baseline: 2.9579x; 1.0012x over previous
import functools
import math

import jax
import jax.numpy as jnp
from jax import lax
from jax.experimental import pallas as pl
from jax.experimental.pallas import tpu as pltpu
from jax.experimental.pallas import tpu_sc as plsc

VOCAB = 100000
D = 128
BATCH = 4096
SEQ = 50
TOTAL = BATCH * SEQ
NW = 32
ROWS_PER_W = TOTAL // NW
CHUNK = 64
N_CHUNKS = ROWS_PER_W // CHUNK
NBUF = 4
N_GROUPS = N_CHUNKS // NBUF
LANES = 16
SCALE = math.sqrt(D)

_mesh = plsc.VectorSubcoreMesh(core_axis_name="c", subcore_axis_name="s")


@functools.partial(
    pl.kernel,
    out_type=jax.ShapeDtypeStruct((TOTAL, D), jnp.float32),
    mesh=_mesh,
    scratch_types=[
        pltpu.VMEM((ROWS_PER_W,), jnp.int32),
        pltpu.VMEM((NBUF, CHUNK, D), jnp.float32),
        pltpu.VMEM((NBUF, CHUNK, D), jnp.float32),
    ]
    + [pltpu.SemaphoreType.DMA] * (2 * NBUF),
    compiler_params=pltpu.CompilerParams(use_tc_tiling_on_sc=True),
)
def _embed_sc(idx_hbm, table_hbm, out_hbm, idx_v, gbuf, sbuf, *sems):
    gsem = sems[:NBUF]
    ssem = sems[NBUF:]
    wid = lax.axis_index("s") * 2 + lax.axis_index("c")
    base = wid * ROWS_PER_W
    pltpu.sync_copy(idx_hbm.at[pl.ds(base, ROWS_PER_W)], idx_v)

    def gather_desc(c, b):
        return pltpu.make_async_copy(
            table_hbm.at[idx_v.at[pl.ds(c * CHUNK, CHUNK)]], gbuf.at[b], gsem[b])

    def store_desc(c, b):
        return pltpu.make_async_copy(
            sbuf.at[b], out_hbm.at[pl.ds(base + c * CHUNK, CHUNK)], ssem[b])

    for b in range(NBUF):
        gather_desc(b, b).start()

    def group(g, _):
        for b in range(NBUF):
            c = g * NBUF + b
            gather_desc(c, b).wait()

            @pl.when(g > 0)
            def _():
                store_desc(c - NBUF, b).wait()

            def scale_row(r, _):
                for j in range(D // LANES):
                    sl = pl.ds(j * LANES, LANES)
                    sbuf[b, r, sl] = gbuf[b, r, sl] * SCALE
                return 0

            lax.fori_loop(0, CHUNK, scale_row, 0)

            @pl.when(c + NBUF < N_CHUNKS)
            def _():
                gather_desc(c + NBUF, b).start()

            store_desc(c, b).start()
        return 0

    lax.fori_loop(0, N_GROUPS, group, 0)

    for b in range(NBUF):
        store_desc(N_CHUNKS - NBUF + b, b).wait()


def kernel(input_ids, embed_weight):
    idx = input_ids.reshape(TOTAL).astype(jnp.int32)
    out = _embed_sc(idx, embed_weight)
    return out.reshape(BATCH, SEQ, D)


# speedup vs baseline: 5.2022x; 1.7587x over previous
import functools
import math

import jax
import jax.numpy as jnp
from jax import lax
from jax.experimental import pallas as pl
from jax.experimental.pallas import tpu as pltpu
from jax.experimental.pallas import tpu_sc as plsc

VOCAB = 100000
D = 128
BATCH = 4096
SEQ = 50
TOTAL = BATCH * SEQ
NW = 32
ROWS_PER_W = TOTAL // NW
B_PER_W = BATCH // NW
NB = 4
CHUNK = NB * SEQ
GATHER_SPLITS = ((0, 128), (128, 72))
N_CHUNKS = B_PER_W // NB
NBUF = 2
N_GROUPS = N_CHUNKS // NBUF
LANES = 16
SCALE = math.sqrt(D)

_mesh = plsc.VectorSubcoreMesh(core_axis_name="c", subcore_axis_name="s")


@functools.partial(
    pl.kernel,
    out_type=jax.ShapeDtypeStruct((BATCH, SEQ, D), jnp.float32),
    mesh=_mesh,
    scratch_types=[
        pltpu.VMEM((ROWS_PER_W,), jnp.int32),
        pltpu.VMEM((NBUF, CHUNK, D), jnp.float32),
        pltpu.VMEM((NBUF, NB, SEQ, D), jnp.float32),
    ]
    + [pltpu.SemaphoreType.DMA] * (2 * NBUF),
)
def _embed_sc(idx_hbm, table_hbm, out_hbm, idx_v, gbuf, sbuf, *sems):
    gsem = sems[:NBUF]
    ssem = sems[NBUF:]
    wid = lax.axis_index("s") * 2 + lax.axis_index("c")
    base = wid * ROWS_PER_W
    b_base = wid * B_PER_W
    pltpu.sync_copy(idx_hbm.at[pl.ds(base, ROWS_PER_W)], idx_v)

    def gather_descs(c, b):
        return [
            pltpu.make_async_copy(
                table_hbm.at[idx_v.at[pl.ds(c * CHUNK + off, n)]],
                gbuf.at[b, pl.ds(off, n)],
                gsem[b],
            )
            for off, n in GATHER_SPLITS
        ]

    def store_desc(c, b):
        return pltpu.make_async_copy(
            sbuf.at[b], out_hbm.at[pl.ds(b_base + c * NB, NB)], ssem[b])

    for b in range(NBUF):
        for d in gather_descs(b, b):
            d.start()

    def group(g, _):
        for b in range(NBUF):
            c = g * NBUF + b
            for d in gather_descs(c, b):
                d.wait()

            @pl.when(g > 0)
            def _():
                store_desc(c - NBUF, b).wait()

            def scale_seq(s, _):
                for bb in range(NB):
                    for j in range(D // LANES):
                        sl = pl.ds(j * LANES, LANES)
                        sbuf[b, bb, s, sl] = gbuf[b, bb * SEQ + s, sl] * SCALE
                return 0

            lax.fori_loop(0, SEQ, scale_seq, 0)

            @pl.when(c + NBUF < N_CHUNKS)
            def _():
                for d in gather_descs(c + NBUF, b):
                    d.start()

            store_desc(c, b).start()
        return 0

    lax.fori_loop(0, N_GROUPS, group, 0)

    for b in range(NBUF):
        store_desc(N_CHUNKS - NBUF + b, b).wait()


def kernel(input_ids, embed_weight):
    idx = input_ids.reshape(TOTAL).astype(jnp.int32)
    return _embed_sc(idx, embed_weight)
